# baseline (speedup 1.0000x reference)
import jax
import jax.numpy as jnp
from jax import lax
from jax.experimental import pallas as pl
from jax.experimental.pallas import tpu as pltpu
from jax.experimental.pallas import tpu_sc as plsc

_D = 64
_NC, _NS = 2, 16
_NW = _NC * _NS
_C = 256
_K = 2
_NBUF = 3


def _build(nsteps):
  mesh = plsc.VectorSubcoreMesh(
      core_axis_name="c", subcore_axis_name="s",
      num_cores=_NC, num_subcores=_NS)
  nchunks = nsteps * _K
  bpw = nchunks * _C

  def body(idx_hbm, table_hbm, out_hbm, idx_v, rows_v,
           g0, g1, g2, w0, w1, w2):
    gsem = [g0, g1, g2]
    wsem = [w0, w1, w2]
    wid = lax.axis_index("s") * _NC + lax.axis_index("c")
    base = wid * bpw
    pltpu.sync_copy(idx_hbm.at[pl.ds(base, bpw)], idx_v)
    out_w = out_hbm.at[pl.ds(base, bpw)]

    def fire_g(j, b):
      for jj in range(_K):
        pltpu.async_copy(table_hbm.at[idx_v.at[pl.ds((j * _K + jj) * _C, _C)]],
                         rows_v.at[b, pl.ds(jj * _C, _C)], gsem[b])

    def wait_g(b):
      pltpu.make_async_copy(out_w.at[pl.ds(0, _K * _C)],
                            rows_v.at[b], gsem[b]).wait()

    def fire_w(i, b):
      pltpu.async_copy(rows_v.at[b], out_w.at[pl.ds(i * _K * _C, _K * _C)],
                       wsem[b])

    def wait_w(b):
      pltpu.make_async_copy(rows_v.at[b], out_w.at[pl.ds(0, _K * _C)],
                            wsem[b]).wait()

    fire_g(0, 0)
    fire_g(1, 1)

    wait_g(0)
    fire_w(0, 0)
    fire_g(2, 2)

    for i in (1, 2):
      b = i % _NBUF
      rb = (i + 2) % _NBUF
      wait_g(b)
      fire_w(i, b)
      wait_w(rb)
      fire_g(i + 2, rb)

    @pl.loop(3, nsteps - 2, step=_NBUF)
    def _mid(t):
      for db in range(_NBUF):
        i = t + db
        b = db
        rb = (db + 2) % _NBUF
        wait_g(b)
        fire_w(i, b)
        wait_w(rb)
        fire_g(i + 2, rb)

    for i in (nsteps - 2, nsteps - 1):
      b = i % _NBUF
      wait_g(b)
      fire_w(i, b)

    for b in range(_NBUF):
      wait_w(b)

  return pl.kernel(
      body,
      out_type=jax.ShapeDtypeStruct((_NW * bpw, _D), jnp.float32),
      mesh=mesh,
      scratch_types=[
          pltpu.VMEM((bpw,), jnp.int32),
          pltpu.VMEM((_NBUF, _K * _C, _D), jnp.float32),
          pltpu.SemaphoreType.DMA,
          pltpu.SemaphoreType.DMA,
          pltpu.SemaphoreType.DMA,
          pltpu.SemaphoreType.DMA,
          pltpu.SemaphoreType.DMA,
          pltpu.SemaphoreType.DMA,
      ],
      compiler_params=pltpu.CompilerParams(use_tc_tiling_on_sc=False),
  )


def kernel(x, W):
  B, H = x.shape
  n = B * H
  rows_per_step = _K * _C
  nsteps = n // (_NW * rows_per_step)
  idx = x.reshape(n).astype(jnp.int32)
  out = _build(nsteps)(idx, W)
  return out.reshape(B, H, _D)


# speedup vs baseline: 1.0016x; 1.0016x over previous
import jax
import jax.numpy as jnp
from jax import lax
from jax.experimental import pallas as pl
from jax.experimental.pallas import tpu as pltpu
from jax.experimental.pallas import tpu_sc as plsc

_D = 64
_NC, _NS = 2, 16
_NW = _NC * _NS
_C = 128
_K = 4
_NBUF = 3


def _build(nsteps):
  mesh = plsc.VectorSubcoreMesh(
      core_axis_name="c", subcore_axis_name="s",
      num_cores=_NC, num_subcores=_NS)
  nchunks = nsteps * _K
  bpw = nchunks * _C

  def body(idx_hbm, table_hbm, out_hbm, idx_v, rows_v,
           g0, g1, g2, w0, w1, w2):
    gsem = [g0, g1, g2]
    wsem = [w0, w1, w2]
    wid = lax.axis_index("s") * _NC + lax.axis_index("c")
    base = wid * bpw
    pltpu.sync_copy(idx_hbm.at[pl.ds(base, bpw)], idx_v)
    out_w = out_hbm.at[pl.ds(base, bpw)]

    def fire_g(j, b):
      for jj in range(_K):
        pltpu.async_copy(table_hbm.at[idx_v.at[pl.ds((j * _K + jj) * _C, _C)]],
                         rows_v.at[b, pl.ds(jj * _C, _C)], gsem[b])

    def wait_g(b):
      pltpu.make_async_copy(out_w.at[pl.ds(0, _K * _C)],
                            rows_v.at[b], gsem[b]).wait()

    def fire_w(i, b):
      pltpu.async_copy(rows_v.at[b], out_w.at[pl.ds(i * _K * _C, _K * _C)],
                       wsem[b])

    def wait_w(b):
      pltpu.make_async_copy(rows_v.at[b], out_w.at[pl.ds(0, _K * _C)],
                            wsem[b]).wait()

    fire_g(0, 0)
    fire_g(1, 1)

    wait_g(0)
    fire_w(0, 0)
    fire_g(2, 2)

    for i in (1, 2):
      b = i % _NBUF
      rb = (i + 2) % _NBUF
      wait_g(b)
      fire_w(i, b)
      wait_w(rb)
      fire_g(i + 2, rb)

    @pl.loop(3, nsteps - 2, step=_NBUF)
    def _mid(t):
      for db in range(_NBUF):
        i = t + db
        b = db
        rb = (db + 2) % _NBUF
        wait_g(b)
        fire_w(i, b)
        wait_w(rb)
        fire_g(i + 2, rb)

    for i in (nsteps - 2, nsteps - 1):
      b = i % _NBUF
      wait_g(b)
      fire_w(i, b)

    for b in range(_NBUF):
      wait_w(b)

  return pl.kernel(
      body,
      out_type=jax.ShapeDtypeStruct((_NW * bpw, _D), jnp.float32),
      mesh=mesh,
      scratch_types=[
          pltpu.VMEM((bpw,), jnp.int32),
          pltpu.VMEM((_NBUF, _K * _C, _D), jnp.float32),
          pltpu.SemaphoreType.DMA,
          pltpu.SemaphoreType.DMA,
          pltpu.SemaphoreType.DMA,
          pltpu.SemaphoreType.DMA,
          pltpu.SemaphoreType.DMA,
          pltpu.SemaphoreType.DMA,
      ],
      compiler_params=pltpu.CompilerParams(use_tc_tiling_on_sc=False),
  )


def kernel(x, W):
  B, H = x.shape
  n = B * H
  rows_per_step = _K * _C
  nsteps = n // (_NW * rows_per_step)
  idx = x.reshape(n).astype(jnp.int32)
  out = _build(nsteps)(idx, W)
  return out.reshape(B, H, _D)


# speedup vs baseline: 1.3281x; 1.3260x over previous
import jax
import jax.numpy as jnp
from jax import lax
from jax.experimental import pallas as pl
from jax.experimental.pallas import tpu as pltpu
from jax.experimental.pallas import tpu_sc as plsc

_D = 64
_NC, _NS = 2, 16
_NW = _NC * _NS
_C = 128
_K = 4
_NBUF = 3


def _build(nsteps):
  mesh = plsc.VectorSubcoreMesh(
      core_axis_name="c", subcore_axis_name="s",
      num_cores=_NC, num_subcores=_NS)
  nchunks = nsteps * _K
  bpw = nchunks * _C

  def body(idx_hbm, table_hbm, out_hbm, idx_v, rows_v,
           g0, g1, g2, w0, w1, w2):
    gsem = [g0, g1, g2]
    wsem = [w0, w1, w2]
    wid = lax.axis_index("s") * _NC + lax.axis_index("c")
    base = wid * bpw
    pltpu.sync_copy(idx_hbm.at[pl.ds(base, bpw)], idx_v)
    out_w = out_hbm.at[pl.ds(base, bpw)]

    def fire_g(j, b):
      for jj in range(_K):
        pltpu.async_copy(table_hbm.at[idx_v.at[pl.ds((j * _K + jj) * _C, _C)]],
                         rows_v.at[b, pl.ds(jj * _C, _C)], gsem[b])

    def wait_g(b):
      pltpu.make_async_copy(out_w.at[pl.ds(0, _K * _C), pl.ds(0, _D)],
                            rows_v.at[b], gsem[b]).wait()

    def fire_w(i, b):
      pltpu.async_copy(rows_v.at[b],
                       out_w.at[pl.ds(i * _K * _C, _K * _C), pl.ds(0, _D)],
                       wsem[b])

    def wait_w(b):
      pltpu.make_async_copy(rows_v.at[b],
                            out_w.at[pl.ds(0, _K * _C), pl.ds(0, _D)],
                            wsem[b]).wait()

    fire_g(0, 0)
    fire_g(1, 1)

    wait_g(0)
    fire_w(0, 0)
    fire_g(2, 2)

    for i in (1, 2):
      b = i % _NBUF
      rb = (i + 2) % _NBUF
      wait_g(b)
      fire_w(i, b)
      wait_w(rb)
      fire_g(i + 2, rb)

    @pl.loop(3, nsteps - 2, step=_NBUF)
    def _mid(t):
      for db in range(_NBUF):
        i = t + db
        b = db
        rb = (db + 2) % _NBUF
        wait_g(b)
        fire_w(i, b)
        wait_w(rb)
        fire_g(i + 2, rb)

    for i in (nsteps - 2, nsteps - 1):
      b = i % _NBUF
      wait_g(b)
      fire_w(i, b)

    for b in range(_NBUF):
      wait_w(b)

  return pl.kernel(
      body,
      out_type=jax.ShapeDtypeStruct((_NW * bpw, 2 * _D), jnp.float32),
      mesh=mesh,
      scratch_types=[
          pltpu.VMEM((bpw,), jnp.int32),
          pltpu.VMEM((_NBUF, _K * _C, _D), jnp.float32),
          pltpu.SemaphoreType.DMA,
          pltpu.SemaphoreType.DMA,
          pltpu.SemaphoreType.DMA,
          pltpu.SemaphoreType.DMA,
          pltpu.SemaphoreType.DMA,
          pltpu.SemaphoreType.DMA,
      ],
      compiler_params=pltpu.CompilerParams(use_tc_tiling_on_sc=False),
  )


def kernel(x, W):
  B, H = x.shape
  n = B * H
  rows_per_step = _K * _C
  nsteps = n // (_NW * rows_per_step)
  idx = x.reshape(n).astype(jnp.int32)
  out = _build(nsteps)(idx, W)
  return out[:, :_D].reshape(B, H, _D)
